# Initial kernel scaffold; baseline (speedup 1.0000x reference)
import jax
import jax.numpy as jnp
from jax.experimental import pallas as pl


def kernel(x, edge_index, W1, b1, W2, b2, Wfc, bfc):
    raise NotImplementedError("write your pallas kernel here")


# speedup vs baseline: 20.6228x; 20.6228x over previous
import functools

import jax
import jax.numpy as jnp
from jax import lax
from jax.experimental import pallas as pl
from jax.experimental.pallas import tpu as pltpu
from jax.experimental.pallas import tpu_sc as plsc

N = 10000
E = 320000
NC = 2
NS = 16
NW = NC * NS
EPT = E // NW
C = 80
NCH = EPT // C
NPAD = 10240
NPT = NPAD // NS
ZR = 128


def _zero_fill(zbuf, acc, s, d):
    def zrow(r, _):
        for kk in range(d // 16):
            zbuf[r, pl.ds(kk * 16, 16)] = jnp.zeros((16,), jnp.float32)
        return 0

    lax.fori_loop(0, ZR, zrow, 0)

    def zcp(t, _):
        pltpu.sync_copy(zbuf, acc.at[pl.ds(s * NPT + t * ZR, ZR)])
        return 0

    lax.fori_loop(0, NPT // ZR, zcp, 0)


def _make_deg():
    mesh = plsc.VectorSubcoreMesh(core_axis_name="c", subcore_axis_name="s")

    @functools.partial(
        pl.kernel,
        out_type=jax.ShapeDtypeStruct((NC * NPAD, 16), jnp.float32),
        mesh=mesh,
        scratch_types=[
            pltpu.VMEM((NCH, C), jnp.int32),
            pltpu.VMEM((C, 16), jnp.float32),
            pltpu.VMEM((ZR, 16), jnp.float32),
            pltpu.VMEM_SHARED((NPAD, 16), jnp.float32),
        ],
        compiler_params=pltpu.CompilerParams(use_tc_tiling_on_sc=False),
    )
    def deg(dst_hbm, out_hbm, dst_v, ones_v, zbuf, acc):
        c = lax.axis_index("c")
        s = lax.axis_index("s")
        wid = s * NC + c
        pltpu.sync_copy(dst_hbm.at[wid], dst_v)

        def orow(r, _):
            ones_v[r, :] = jnp.ones((16,), jnp.float32)
            return 0

        lax.fori_loop(0, C, orow, 0)
        _zero_fill(zbuf, acc, s, 16)
        plsc.subcore_barrier()

        def chunk(j, _):
            pltpu.sync_copy(ones_v, acc.at[dst_v.at[j]], add=True)
            return 0

        lax.fori_loop(0, NCH, chunk, 0)
        plsc.subcore_barrier()
        pltpu.sync_copy(
            acc.at[pl.ds(s * NPT, NPT)],
            out_hbm.at[pl.ds(c * NPAD + s * NPT, NPT)],
        )

    return deg


def _make_agg(d):
    mesh = plsc.VectorSubcoreMesh(core_axis_name="c", subcore_axis_name="s")

    @functools.partial(
        pl.kernel,
        out_type=jax.ShapeDtypeStruct((NC * NPAD, d), jnp.float32),
        mesh=mesh,
        scratch_types=[
            pltpu.VMEM((NCH, C), jnp.int32),
            pltpu.VMEM((NCH, C), jnp.int32),
            pltpu.VMEM((C, d), jnp.float32),
            pltpu.VMEM((ZR, d), jnp.float32),
            pltpu.VMEM_SHARED((NPAD, d), jnp.float32),
            pltpu.SemaphoreType.DMA,
        ],
        compiler_params=pltpu.CompilerParams(use_tc_tiling_on_sc=False),
    )
    def agg(hs_hbm, src_hbm, dst_hbm, out_hbm, src_v, dst_v, buf, zbuf, acc, sem):
        c = lax.axis_index("c")
        s = lax.axis_index("s")
        wid = s * NC + c
        pltpu.sync_copy(src_hbm.at[wid], src_v)
        pltpu.sync_copy(dst_hbm.at[wid], dst_v)
        _zero_fill(zbuf, acc, s, d)
        plsc.subcore_barrier()

        def chunk(j, _):
            pltpu.async_copy(hs_hbm.at[src_v.at[j]], buf, sem).wait()
            pltpu.sync_copy(buf, acc.at[dst_v.at[j]], add=True)
            return 0

        lax.fori_loop(0, NCH, chunk, 0)
        plsc.subcore_barrier()
        pltpu.sync_copy(
            acc.at[pl.ds(s * NPT, NPT)],
            out_hbm.at[pl.ds(c * NPAD + s * NPT, NPT)],
        )

    return agg


_deg_kernel = _make_deg()
_agg64 = _make_agg(64)
_agg128 = _make_agg(128)


_R = 1000
_G = N // _R


def _tc1_body(c0, c1, x_ref, w1, dinv_ref, hs_ref):
    deg = c0[:, 0:1] + c1[:, 0:1] + 1.0
    dinv = lax.rsqrt(deg)
    h = jnp.dot(x_ref[...], w1[...], preferred_element_type=jnp.float32)
    dinv_ref[...] = dinv
    hs_ref[...] = h * dinv


def _tc1(c0, c1, x, w1):
    return pl.pallas_call(
        _tc1_body,
        grid=(_G,),
        in_specs=[
            pl.BlockSpec((_R, 16), lambda i: (i, 0)),
            pl.BlockSpec((_R, 16), lambda i: (i, 0)),
            pl.BlockSpec((_R, 128), lambda i: (i, 0)),
            pl.BlockSpec((128, 64), lambda i: (0, 0)),
        ],
        out_specs=[
            pl.BlockSpec((_R, 1), lambda i: (i, 0)),
            pl.BlockSpec((_R, 64), lambda i: (i, 0)),
        ],
        out_shape=[
            jax.ShapeDtypeStruct((N, 1), jnp.float32),
            jax.ShapeDtypeStruct((N, 64), jnp.float32),
        ],
    )(c0, c1, x, w1)


def _tc2_body(u0, u1, hs, dinv_ref, b1, w2, hs2_ref):
    dinv = dinv_ref[...]
    a = dinv * (u0[...] + u1[...] + hs[...]) + b1[...]
    r = jnp.maximum(a, 0.0)
    h2 = jnp.dot(r, w2[...], preferred_element_type=jnp.float32)
    hs2_ref[...] = h2 * dinv


def _tc2(ua, ub, hs1, dinv, b1, w2):
    return pl.pallas_call(
        _tc2_body,
        grid=(_G,),
        in_specs=[
            pl.BlockSpec((_R, 64), lambda i: (i, 0)),
            pl.BlockSpec((_R, 64), lambda i: (i, 0)),
            pl.BlockSpec((_R, 64), lambda i: (i, 0)),
            pl.BlockSpec((_R, 1), lambda i: (i, 0)),
            pl.BlockSpec((1, 64), lambda i: (0, 0)),
            pl.BlockSpec((64, 128), lambda i: (0, 0)),
        ],
        out_specs=pl.BlockSpec((_R, 128), lambda i: (i, 0)),
        out_shape=jax.ShapeDtypeStruct((N, 128), jnp.float32),
    )(ua, ub, hs1, dinv, b1, w2)


def _tc3_body(u0, u1, hs, dinv_ref, b2, wfc, bfc, out_ref):
    dinv = dinv_ref[...]
    a = dinv * (u0[...] + u1[...] + hs[...]) + b2[...]
    r = jnp.maximum(a, 0.0)
    out_ref[...] = jnp.dot(r, wfc[...], preferred_element_type=jnp.float32) + bfc[...]


def _tc3(ua, ub, hs2, dinv, b2, wfc, bfc):
    return pl.pallas_call(
        _tc3_body,
        grid=(_G,),
        in_specs=[
            pl.BlockSpec((_R, 128), lambda i: (i, 0)),
            pl.BlockSpec((_R, 128), lambda i: (i, 0)),
            pl.BlockSpec((_R, 128), lambda i: (i, 0)),
            pl.BlockSpec((_R, 1), lambda i: (i, 0)),
            pl.BlockSpec((1, 128), lambda i: (0, 0)),
            pl.BlockSpec((128, 1), lambda i: (0, 0)),
            pl.BlockSpec((1, 1), lambda i: (0, 0)),
        ],
        out_specs=pl.BlockSpec((_R, 1), lambda i: (i, 0)),
        out_shape=jax.ShapeDtypeStruct((N, 1), jnp.float32),
    )(ua, ub, hs2, dinv, b2, wfc, bfc)


def kernel(x, edge_index, W1, b1, W2, b2, Wfc, bfc):
    ei = edge_index.astype(jnp.int32)
    src3 = ei[0].reshape(NW, NCH, C)
    dst3 = ei[1].reshape(NW, NCH, C)

    counts = _deg_kernel(dst3)
    dinv, hs1 = _tc1(counts[:N], counts[NPAD:NPAD + N], x, W1)
    u1 = _agg64(hs1, src3, dst3)
    hs2 = _tc2(u1[:N], u1[NPAD:NPAD + N], hs1, dinv, b1.reshape(1, 64), W2)
    u2 = _agg128(hs2, src3, dst3)
    out = _tc3(u2[:N], u2[NPAD:NPAD + N], hs2, dinv, b2.reshape(1, 128), Wfc, bfc.reshape(1, 1))
    return out


# speedup vs baseline: 23.2978x; 1.1297x over previous
import functools

import jax
import jax.numpy as jnp
from jax import lax
from jax.experimental import pallas as pl
from jax.experimental.pallas import tpu as pltpu
from jax.experimental.pallas import tpu_sc as plsc

N = 10000
E = 320000
NC = 2
NS = 16
NW = NC * NS
C = 80
EPW = E // NW
NCHW = EPW // C
EPS = E // NS
NCHS = EPS // C
NPAD = 10240
NPT = NPAD // NS
ZR = 128


def _zero_fill(zbuf, acc, s, d):
    def zrow(r, _):
        for kk in range(d // 16):
            zbuf[r, pl.ds(kk * 16, 16)] = jnp.zeros((16,), jnp.float32)
        return 0

    lax.fori_loop(0, ZR, zrow, 0)

    def zcp(t, _):
        pltpu.sync_copy(zbuf, acc.at[pl.ds(s * NPT + t * ZR, ZR)])
        return 0

    lax.fori_loop(0, NPT // ZR, zcp, 0)


def _make_deg():
    mesh = plsc.VectorSubcoreMesh(core_axis_name="c", subcore_axis_name="s")

    @functools.partial(
        pl.kernel,
        out_type=jax.ShapeDtypeStruct((NC * NPAD, 16), jnp.float32),
        mesh=mesh,
        scratch_types=[
            pltpu.VMEM((NCHW, C), jnp.int32),
            pltpu.VMEM((C, 16), jnp.float32),
            pltpu.VMEM((ZR, 16), jnp.float32),
            pltpu.VMEM_SHARED((NPAD, 16), jnp.float32),
        ],
        compiler_params=pltpu.CompilerParams(use_tc_tiling_on_sc=False),
    )
    def deg(dst_hbm, out_hbm, dst_v, ones_v, zbuf, acc):
        c = lax.axis_index("c")
        s = lax.axis_index("s")
        wid = s * NC + c
        pltpu.sync_copy(dst_hbm.at[wid], dst_v)

        def orow(r, _):
            ones_v[r, :] = jnp.ones((16,), jnp.float32)
            return 0

        lax.fori_loop(0, C, orow, 0)
        _zero_fill(zbuf, acc, s, 16)
        plsc.subcore_barrier()

        def chunk(j, _):
            pltpu.sync_copy(ones_v, acc.at[dst_v.at[j]], add=True)
            return 0

        lax.fori_loop(0, NCHW, chunk, 0)
        plsc.subcore_barrier()
        pltpu.sync_copy(
            acc.at[pl.ds(s * NPT, NPT)],
            out_hbm.at[pl.ds(c * NPAD + s * NPT, NPT)],
        )

    return deg


def _make_agg(dh):
    mesh = plsc.VectorSubcoreMesh(core_axis_name="c", subcore_axis_name="s")

    @functools.partial(
        pl.kernel,
        out_type=jax.ShapeDtypeStruct((NC * NPAD, dh), jnp.float32),
        mesh=mesh,
        scratch_types=[
            pltpu.VMEM((NCHS, C), jnp.int32),
            pltpu.VMEM((NCHS, C), jnp.int32),
            pltpu.VMEM((C, dh), jnp.float32),
            pltpu.VMEM((C, dh), jnp.float32),
            pltpu.VMEM((ZR, dh), jnp.float32),
            pltpu.VMEM_SHARED((NPAD, dh), jnp.float32),
            pltpu.SemaphoreType.DMA,
            pltpu.SemaphoreType.DMA,
        ],
        compiler_params=pltpu.CompilerParams(use_tc_tiling_on_sc=False),
    )
    def agg(hsa_hbm, hsb_hbm, src_hbm, dst_hbm, out_hbm,
            src_v, dst_v, buf_a, buf_b, zbuf, acc, sem_a, sem_b):
        c = lax.axis_index("c")
        s = lax.axis_index("s")
        pltpu.sync_copy(src_hbm.at[s], src_v)
        pltpu.sync_copy(dst_hbm.at[s], dst_v)
        _zero_fill(zbuf, acc, s, dh)
        plsc.subcore_barrier()

        def run(hs_hbm):
            def fire(j, buf, sem):
                pltpu.async_copy(hs_hbm.at[src_v.at[j]], buf, sem)

            def drain(j, buf, sem):
                pltpu.make_async_copy(hs_hbm.at[src_v.at[j]], buf, sem).wait()
                pltpu.sync_copy(buf, acc.at[dst_v.at[j]], add=True)

            fire(0, buf_a, sem_a)
            fire(1, buf_b, sem_b)

            def pair(i, _):
                drain(2 * i, buf_a, sem_a)
                fire(2 * i + 2, buf_a, sem_a)
                drain(2 * i + 1, buf_b, sem_b)
                fire(2 * i + 3, buf_b, sem_b)
                return 0

            lax.fori_loop(0, NCHS // 2 - 1, pair, 0)
            drain(NCHS - 2, buf_a, sem_a)
            drain(NCHS - 1, buf_b, sem_b)

        @pl.when(c == 0)
        def _():
            run(hsa_hbm)

        @pl.when(c == 1)
        def _():
            run(hsb_hbm)

        plsc.subcore_barrier()
        pltpu.sync_copy(
            acc.at[pl.ds(s * NPT, NPT)],
            out_hbm.at[pl.ds(c * NPAD + s * NPT, NPT)],
        )

    return agg


_deg_kernel = _make_deg()
_agg32 = _make_agg(32)
_agg64 = _make_agg(64)


_R = 1000
_G = N // _R


def _tc1_body(c0, c1, x_ref, w1, dinv_ref, hsa_ref, hsb_ref):
    deg = c0[:, 0:1] + c1[:, 0:1] + 1.0
    dinv = lax.rsqrt(deg)
    h = jnp.dot(x_ref[...], w1[...], preferred_element_type=jnp.float32)
    hs = h * dinv
    dinv_ref[...] = dinv
    hsa_ref[...] = hs[:, :32]
    hsb_ref[...] = hs[:, 32:]


def _tc1(c0, c1, x, w1):
    return pl.pallas_call(
        _tc1_body,
        grid=(_G,),
        in_specs=[
            pl.BlockSpec((_R, 16), lambda i: (i, 0)),
            pl.BlockSpec((_R, 16), lambda i: (i, 0)),
            pl.BlockSpec((_R, 128), lambda i: (i, 0)),
            pl.BlockSpec((128, 64), lambda i: (0, 0)),
        ],
        out_specs=[
            pl.BlockSpec((_R, 1), lambda i: (i, 0)),
            pl.BlockSpec((_R, 32), lambda i: (i, 0)),
            pl.BlockSpec((_R, 32), lambda i: (i, 0)),
        ],
        out_shape=[
            jax.ShapeDtypeStruct((N, 1), jnp.float32),
            jax.ShapeDtypeStruct((N, 32), jnp.float32),
            jax.ShapeDtypeStruct((N, 32), jnp.float32),
        ],
    )(c0, c1, x, w1)


def _tc2_body(ua, ub, hsa, hsb, dinv_ref, b1, w2, hs2a_ref, hs2b_ref):
    dinv = dinv_ref[...]
    u = jnp.concatenate([ua[...], ub[...]], axis=1)
    hs = jnp.concatenate([hsa[...], hsb[...]], axis=1)
    a = dinv * (u + hs) + b1[...]
    r = jnp.maximum(a, 0.0)
    h2 = jnp.dot(r, w2[...], preferred_element_type=jnp.float32)
    hs2 = h2 * dinv
    hs2a_ref[...] = hs2[:, :64]
    hs2b_ref[...] = hs2[:, 64:]


def _tc2(ua, ub, hsa, hsb, dinv, b1, w2):
    return pl.pallas_call(
        _tc2_body,
        grid=(_G,),
        in_specs=[
            pl.BlockSpec((_R, 32), lambda i: (i, 0)),
            pl.BlockSpec((_R, 32), lambda i: (i, 0)),
            pl.BlockSpec((_R, 32), lambda i: (i, 0)),
            pl.BlockSpec((_R, 32), lambda i: (i, 0)),
            pl.BlockSpec((_R, 1), lambda i: (i, 0)),
            pl.BlockSpec((1, 64), lambda i: (0, 0)),
            pl.BlockSpec((64, 128), lambda i: (0, 0)),
        ],
        out_specs=[
            pl.BlockSpec((_R, 64), lambda i: (i, 0)),
            pl.BlockSpec((_R, 64), lambda i: (i, 0)),
        ],
        out_shape=[
            jax.ShapeDtypeStruct((N, 64), jnp.float32),
            jax.ShapeDtypeStruct((N, 64), jnp.float32),
        ],
    )(ua, ub, hsa, hsb, dinv, b1, w2)


def _tc3_body(ua, ub, hsa, hsb, dinv_ref, b2, wfc, bfc, out_ref):
    dinv = dinv_ref[...]
    u = jnp.concatenate([ua[...], ub[...]], axis=1)
    hs = jnp.concatenate([hsa[...], hsb[...]], axis=1)
    a = dinv * (u + hs) + b2[...]
    r = jnp.maximum(a, 0.0)
    out_ref[...] = jnp.dot(r, wfc[...], preferred_element_type=jnp.float32) + bfc[...]


def _tc3(ua, ub, hsa, hsb, dinv, b2, wfc, bfc):
    return pl.pallas_call(
        _tc3_body,
        grid=(_G,),
        in_specs=[
            pl.BlockSpec((_R, 64), lambda i: (i, 0)),
            pl.BlockSpec((_R, 64), lambda i: (i, 0)),
            pl.BlockSpec((_R, 64), lambda i: (i, 0)),
            pl.BlockSpec((_R, 64), lambda i: (i, 0)),
            pl.BlockSpec((_R, 1), lambda i: (i, 0)),
            pl.BlockSpec((1, 128), lambda i: (0, 0)),
            pl.BlockSpec((128, 1), lambda i: (0, 0)),
            pl.BlockSpec((1, 1), lambda i: (0, 0)),
        ],
        out_specs=pl.BlockSpec((_R, 1), lambda i: (i, 0)),
        out_shape=jax.ShapeDtypeStruct((N, 1), jnp.float32),
    )(ua, ub, hsa, hsb, dinv, b2, wfc, bfc)


def kernel(x, edge_index, W1, b1, W2, b2, Wfc, bfc):
    ei = edge_index.astype(jnp.int32)
    src_w = ei[0].reshape(NW, NCHW, C)
    dst_w = ei[1].reshape(NW, NCHW, C)
    src_s = ei[0].reshape(NS, NCHS, C)
    dst_s = ei[1].reshape(NS, NCHS, C)

    counts = _deg_kernel(dst_w)
    dinv, hsa, hsb = _tc1(counts[:N], counts[NPAD:NPAD + N], x, W1)
    u1 = _agg32(hsa, hsb, src_s, dst_s)
    hs2a, hs2b = _tc2(u1[:N], u1[NPAD:NPAD + N], hsa, hsb, dinv,
                      b1.reshape(1, 64), W2)
    u2 = _agg64(hs2a, hs2b, src_s, dst_s)
    out = _tc3(u2[:N], u2[NPAD:NPAD + N], hs2a, hs2b, dinv,
               b2.reshape(1, 128), Wfc, bfc.reshape(1, 1))
    return out


# speedup vs baseline: 25.5471x; 1.0965x over previous
import functools

import jax
import jax.numpy as jnp
from jax import lax
from jax.experimental import pallas as pl
from jax.experimental.pallas import tpu as pltpu
from jax.experimental.pallas import tpu_sc as plsc

N = 10000
E = 320000
NC = 2
NS = 16
C = 80
EPS = E // NS
NCHS = EPS // C
NPAD = 10240
NPT = NPAD // NS
ZR = 128


def _zero_fill(zbuf, acc, s, d):
    def zrow(r, _):
        for kk in range(d // 16):
            zbuf[r, pl.ds(kk * 16, 16)] = jnp.zeros((16,), jnp.float32)
        return 0

    lax.fori_loop(0, ZR, zrow, 0)

    def zcp(t, _):
        pltpu.sync_copy(zbuf, acc.at[pl.ds(s * NPT + t * ZR, ZR)])
        return 0

    lax.fori_loop(0, NPT // ZR, zcp, 0)


def _make_deg():
    mesh = plsc.VectorSubcoreMesh(core_axis_name="c", subcore_axis_name="s")

    @functools.partial(
        pl.kernel,
        out_type=jax.ShapeDtypeStruct((NC * NPAD, 16), jnp.float32),
        mesh=mesh,
        scratch_types=[
            pltpu.VMEM((NCHS, C), jnp.int32),
            pltpu.VMEM((C, 16), jnp.float32),
            pltpu.VMEM((ZR, 16), jnp.float32),
            pltpu.VMEM_SHARED((NPAD, 16), jnp.float32),
        ],
        compiler_params=pltpu.CompilerParams(use_tc_tiling_on_sc=False),
    )
    def deg(eidx_hbm, out_hbm, dst_v, ones_v, zbuf, acc):
        c = lax.axis_index("c")
        s = lax.axis_index("s")
        pltpu.sync_copy(eidx_hbm.at[1, s], dst_v)

        def orow(r, _):
            ones_v[r, :] = jnp.ones((16,), jnp.float32)
            return 0

        lax.fori_loop(0, C, orow, 0)
        _zero_fill(zbuf, acc, s, 16)
        plsc.subcore_barrier()

        base = c * (NCHS // 2)

        def chunk(j, _):
            pltpu.sync_copy(ones_v, acc.at[dst_v.at[base + j]], add=True)
            return 0

        lax.fori_loop(0, NCHS // 2, chunk, 0)
        plsc.subcore_barrier()
        pltpu.sync_copy(
            acc.at[pl.ds(s * NPT, NPT)],
            out_hbm.at[pl.ds(c * NPAD + s * NPT, NPT)],
        )

    return deg


def _make_agg(dh):
    mesh = plsc.VectorSubcoreMesh(core_axis_name="c", subcore_axis_name="s")

    @functools.partial(
        pl.kernel,
        out_type=jax.ShapeDtypeStruct((NC * NPAD, dh), jnp.float32),
        mesh=mesh,
        scratch_types=[
            pltpu.VMEM((NCHS, C), jnp.int32),
            pltpu.VMEM((NCHS, C), jnp.int32),
            pltpu.VMEM((C, dh), jnp.float32),
            pltpu.VMEM((C, dh), jnp.float32),
            pltpu.VMEM((ZR, dh), jnp.float32),
            pltpu.VMEM_SHARED((NPAD, dh), jnp.float32),
            pltpu.SemaphoreType.DMA,
            pltpu.SemaphoreType.DMA,
        ],
        compiler_params=pltpu.CompilerParams(use_tc_tiling_on_sc=False),
    )
    def agg(hsa_hbm, hsb_hbm, eidx_hbm, out_hbm,
            src_v, dst_v, buf_a, buf_b, zbuf, acc, sem_a, sem_b):
        c = lax.axis_index("c")
        s = lax.axis_index("s")
        pltpu.sync_copy(eidx_hbm.at[0, s], src_v)
        pltpu.sync_copy(eidx_hbm.at[1, s], dst_v)
        _zero_fill(zbuf, acc, s, dh)
        plsc.subcore_barrier()

        def run(hs_hbm):
            def fire(j, buf, sem):
                pltpu.async_copy(hs_hbm.at[src_v.at[j]], buf, sem)

            def drain(j, buf, sem):
                pltpu.make_async_copy(hs_hbm.at[src_v.at[j]], buf, sem).wait()
                pltpu.sync_copy(buf, acc.at[dst_v.at[j]], add=True)

            fire(0, buf_a, sem_a)
            fire(1, buf_b, sem_b)

            def pair(i, _):
                drain(2 * i, buf_a, sem_a)
                fire(2 * i + 2, buf_a, sem_a)
                drain(2 * i + 1, buf_b, sem_b)
                fire(2 * i + 3, buf_b, sem_b)
                return 0

            lax.fori_loop(0, NCHS // 2 - 1, pair, 0)
            drain(NCHS - 2, buf_a, sem_a)
            drain(NCHS - 1, buf_b, sem_b)

        @pl.when(c == 0)
        def _():
            run(hsa_hbm)

        @pl.when(c == 1)
        def _():
            run(hsb_hbm)

        plsc.subcore_barrier()
        pltpu.sync_copy(
            acc.at[pl.ds(s * NPT, NPT)],
            out_hbm.at[pl.ds(c * NPAD + s * NPT, NPT)],
        )

    return agg


_deg_kernel = _make_deg()
_agg32 = _make_agg(32)
_agg64 = _make_agg(64)


_R = 2560
_G = NPAD // _R


def _tc1_body(c0, c1, x_ref, w1, dinv_ref, hsa_ref, hsb_ref):
    deg = c0[:, 0:1] + c1[:, 0:1] + 1.0
    dinv = lax.rsqrt(deg)
    h = jnp.dot(x_ref[...], w1[...], preferred_element_type=jnp.float32)
    hs = h * dinv
    dinv_ref[...] = dinv
    hsa_ref[...] = hs[:, :32]
    hsb_ref[...] = hs[:, 32:]


def _tc1(counts, x, w1):
    return pl.pallas_call(
        _tc1_body,
        grid=(_G,),
        in_specs=[
            pl.BlockSpec((_R, 16), lambda i: (i, 0)),
            pl.BlockSpec((_R, 16), lambda i: (i + _G, 0)),
            pl.BlockSpec((_R, 128), lambda i: (i, 0)),
            pl.BlockSpec((128, 64), lambda i: (0, 0)),
        ],
        out_specs=[
            pl.BlockSpec((_R, 1), lambda i: (i, 0)),
            pl.BlockSpec((_R, 32), lambda i: (i, 0)),
            pl.BlockSpec((_R, 32), lambda i: (i, 0)),
        ],
        out_shape=[
            jax.ShapeDtypeStruct((NPAD, 1), jnp.float32),
            jax.ShapeDtypeStruct((NPAD, 32), jnp.float32),
            jax.ShapeDtypeStruct((NPAD, 32), jnp.float32),
        ],
    )(counts, counts, x, w1)


def _tc2_body(ua, ub, hsa, hsb, dinv_ref, b1, w2, hs2a_ref, hs2b_ref):
    dinv = dinv_ref[...]
    u = jnp.concatenate([ua[...], ub[...]], axis=1)
    hs = jnp.concatenate([hsa[...], hsb[...]], axis=1)
    a = dinv * (u + hs) + b1[...]
    r = jnp.maximum(a, 0.0)
    h2 = jnp.dot(r, w2[...], preferred_element_type=jnp.float32)
    hs2 = h2 * dinv
    hs2a_ref[...] = hs2[:, :64]
    hs2b_ref[...] = hs2[:, 64:]


def _tc2(u, hsa, hsb, dinv, b1, w2):
    return pl.pallas_call(
        _tc2_body,
        grid=(_G,),
        in_specs=[
            pl.BlockSpec((_R, 32), lambda i: (i, 0)),
            pl.BlockSpec((_R, 32), lambda i: (i + _G, 0)),
            pl.BlockSpec((_R, 32), lambda i: (i, 0)),
            pl.BlockSpec((_R, 32), lambda i: (i, 0)),
            pl.BlockSpec((_R, 1), lambda i: (i, 0)),
            pl.BlockSpec((1, 64), lambda i: (0, 0)),
            pl.BlockSpec((64, 128), lambda i: (0, 0)),
        ],
        out_specs=[
            pl.BlockSpec((_R, 64), lambda i: (i, 0)),
            pl.BlockSpec((_R, 64), lambda i: (i, 0)),
        ],
        out_shape=[
            jax.ShapeDtypeStruct((NPAD, 64), jnp.float32),
            jax.ShapeDtypeStruct((NPAD, 64), jnp.float32),
        ],
    )(u, u, hsa, hsb, dinv, b1, w2)


def _tc3_body(ua, ub, hsa, hsb, dinv_ref, b2, wfc, bfc, out_ref):
    dinv = dinv_ref[...]
    u = jnp.concatenate([ua[...], ub[...]], axis=1)
    hs = jnp.concatenate([hsa[...], hsb[...]], axis=1)
    a = dinv * (u + hs) + b2[...]
    r = jnp.maximum(a, 0.0)
    out_ref[...] = jnp.dot(r, wfc[...], preferred_element_type=jnp.float32) + bfc[...]


def _tc3(u, hsa, hsb, dinv, b2, wfc, bfc):
    return pl.pallas_call(
        _tc3_body,
        grid=(_G,),
        in_specs=[
            pl.BlockSpec((_R, 64), lambda i: (i, 0)),
            pl.BlockSpec((_R, 64), lambda i: (i + _G, 0)),
            pl.BlockSpec((_R, 64), lambda i: (i, 0)),
            pl.BlockSpec((_R, 64), lambda i: (i, 0)),
            pl.BlockSpec((_R, 1), lambda i: (i, 0)),
            pl.BlockSpec((1, 128), lambda i: (0, 0)),
            pl.BlockSpec((128, 1), lambda i: (0, 0)),
            pl.BlockSpec((1, 1), lambda i: (0, 0)),
        ],
        out_specs=pl.BlockSpec((_R, 1), lambda i: (i, 0)),
        out_shape=jax.ShapeDtypeStruct((NPAD, 1), jnp.float32),
    )(u, u, hsa, hsb, dinv, b2, wfc, bfc)


def kernel(x, edge_index, W1, b1, W2, b2, Wfc, bfc):
    eidx = edge_index.astype(jnp.int32).reshape(2, NS, NCHS, C)

    counts = _deg_kernel(eidx)
    dinv, hsa, hsb = _tc1(counts, x, W1)
    u1 = _agg32(hsa, hsb, eidx)
    hs2a, hs2b = _tc2(u1, hsa, hsb, dinv, b1.reshape(1, 64), W2)
    u2 = _agg64(hs2a, hs2b, eidx)
    outp = _tc3(u2, hs2a, hs2b, dinv, b2.reshape(1, 128), Wfc, bfc.reshape(1, 1))
    return outp[:N]
